# baseline (speedup 1.0000x reference)
import functools

import jax
import jax.numpy as jnp
from jax import lax
from jax.experimental import pallas as pl
from jax.experimental.pallas import tpu as pltpu
from jax.experimental.pallas import tpu_sc as plsc


def _sc_gather(idx, table):
    B, = idx.shape
    V, D = table.shape
    info = plsc.get_sparse_core_info()
    NC, NS = info.num_cores, info.num_subcores
    NW = NC * NS
    b_per_w = B // NW
    mesh = plsc.VectorSubcoreMesh(core_axis_name="c", subcore_axis_name="s")

    @functools.partial(
        pl.kernel,
        mesh=mesh,
        out_type=jax.ShapeDtypeStruct((B, D), table.dtype),
        scratch_types=[
            pltpu.VMEM((b_per_w,), jnp.int32),
            pltpu.VMEM((b_per_w, D), table.dtype),
            pltpu.SemaphoreType.DMA,
        ],
    )
    def k(idx_hbm, table_hbm, out_hbm, idx_v, rows_v, sem):
        wid = lax.axis_index("s") * NC + lax.axis_index("c")
        base = wid * b_per_w
        pltpu.sync_copy(idx_hbm.at[pl.ds(base, b_per_w)], idx_v)
        pltpu.async_copy(table_hbm.at[idx_v], rows_v, sem).wait()
        pltpu.sync_copy(rows_v, out_hbm.at[pl.ds(base, b_per_w)])

    return k(idx, table)


def _tc_tail(z, wt, bt, V, v_al):
    B, D = z.shape
    tail_c = V - v_al

    def body(z_ref, wt_ref, bt_ref, o_ref):
        o_ref[:, :tail_c] = lax.dot_general(
            z_ref[...], wt_ref[...],
            dimension_numbers=(((1,), (1,)), ((), ())),
            preferred_element_type=jnp.float32,
        ) + bt_ref[...]

    return pl.pallas_call(
        body,
        grid=(1,),
        in_specs=[
            pl.BlockSpec((B, D), lambda i: (0, 0)),
            pl.BlockSpec((tail_c, D), lambda i: (0, 0)),
            pl.BlockSpec((1, tail_c), lambda i: (0, 0)),
        ],
        out_specs=pl.BlockSpec((B, 128), lambda i: (0, v_al // 128)),
        out_shape=jax.ShapeDtypeStruct((B, V), jnp.float32),
    )(z, wt, bt)


def _tc_project(z, out_w, out_b, tile, nq):
    B, D = z.shape
    V, _ = out_w.shape
    v_al = (V // 128) * 128
    n_full = v_al // tile
    last = v_al - n_full * tile
    n = n_full + (1 if last else 0)
    rows_q = B // nq
    assert v_al % 128 == 0 and last % 128 == 0 and B % nq == 0

    out0 = _tc_tail(z, out_w[v_al:], out_b[v_al:].reshape(1, V - v_al),
                    V, v_al)

    def body(o_in, z_ref, w_ref, b_ref, o_hbm, acc_ref, sems):
        del o_in
        i = pl.program_id(0)
        slot = lax.rem(i, 2)

        @pl.when(i >= 2)
        def _wait_prev():
            for q in range(nq):
                pltpu.make_async_copy(
                    acc_ref.at[slot, pl.ds(q * rows_q, rows_q), :],
                    o_hbm.at[pl.ds(q * rows_q, rows_q),
                             pl.ds((i - 2) * tile, tile)],
                    sems.at[slot, q]).wait()

        acc_ref[slot] = lax.dot_general(
            z_ref[...], w_ref[...],
            dimension_numbers=(((1,), (1,)), ((), ())),
            preferred_element_type=jnp.float32,
        ) + b_ref[...]

        @pl.when(i < n_full)
        def _issue_full():
            for q in range(nq):
                pltpu.make_async_copy(
                    acc_ref.at[slot, pl.ds(q * rows_q, rows_q), :],
                    o_hbm.at[pl.ds(q * rows_q, rows_q),
                             pl.ds(i * tile, tile)],
                    sems.at[slot, q]).start()

        if last:
            @pl.when(i == n - 1)
            def _issue_last():
                for q in range(nq):
                    pltpu.make_async_copy(
                        acc_ref.at[slot, pl.ds(q * rows_q, rows_q),
                                   pl.ds(0, last)],
                        o_hbm.at[pl.ds(q * rows_q, rows_q),
                                 pl.ds(i * tile, last)],
                        sems.at[slot, q]).start()

        @pl.when(i == n - 1)
        def _drain():
            lw = last if last else tile
            for q in range(nq):
                pltpu.make_async_copy(
                    acc_ref.at[1 - slot, pl.ds(q * rows_q, rows_q), :],
                    o_hbm.at[pl.ds(q * rows_q, rows_q),
                             pl.ds((i - 1) * tile, tile)],
                    sems.at[1 - slot, q]).wait()
                pltpu.make_async_copy(
                    acc_ref.at[slot, pl.ds(q * rows_q, rows_q),
                               pl.ds(0, lw)],
                    o_hbm.at[pl.ds(q * rows_q, rows_q),
                             pl.ds(i * tile, lw)],
                    sems.at[slot, q]).wait()

    return pl.pallas_call(
        body,
        grid=(n,),
        in_specs=[
            pl.BlockSpec(memory_space=pl.ANY),
            pl.BlockSpec((B, D), lambda i: (0, 0)),
            pl.BlockSpec((tile, D), lambda i: (i, 0)),
            pl.BlockSpec((1, tile), lambda i: (0, i)),
        ],
        out_specs=pl.BlockSpec(memory_space=pl.ANY),
        out_shape=jax.ShapeDtypeStruct((B, V), jnp.float32),
        input_output_aliases={0: 0},
        scratch_shapes=[
            pltpu.VMEM((2, B, tile), jnp.float32),
            pltpu.SemaphoreType.DMA((2, nq)),
        ],
    )(out0, z, out_w, out_b.reshape(1, V))


def kernel(x, emb_table, out_w, out_b):
    z = _sc_gather(x.astype(jnp.int32), emb_table)
    return _tc_project(z, out_w, out_b, tile=2048, nq=4)


# speedup vs baseline: 2.3477x; 2.3477x over previous
import functools

import jax
import jax.numpy as jnp
from jax import lax
from jax.experimental import pallas as pl
from jax.experimental.pallas import tpu as pltpu
from jax.experimental.pallas import tpu_sc as plsc


def _sc_gather(idx, table):
    B, = idx.shape
    V, D = table.shape
    info = plsc.get_sparse_core_info()
    NC, NS = info.num_cores, info.num_subcores
    NW = NC * NS
    b_per_w = B // NW
    mesh = plsc.VectorSubcoreMesh(core_axis_name="c", subcore_axis_name="s")

    @functools.partial(
        pl.kernel,
        mesh=mesh,
        out_type=jax.ShapeDtypeStruct((B, D), table.dtype),
        scratch_types=[
            pltpu.VMEM((b_per_w,), jnp.int32),
            pltpu.VMEM((b_per_w, D), table.dtype),
            pltpu.SemaphoreType.DMA,
        ],
    )
    def k(idx_hbm, table_hbm, out_hbm, idx_v, rows_v, sem):
        wid = lax.axis_index("s") * NC + lax.axis_index("c")
        base = wid * b_per_w
        pltpu.sync_copy(idx_hbm.at[pl.ds(base, b_per_w)], idx_v)
        pltpu.async_copy(table_hbm.at[idx_v], rows_v, sem).wait()
        pltpu.sync_copy(rows_v, out_hbm.at[pl.ds(base, b_per_w)])

    return k(idx, table)


def _mm_t_body(z_ref, w_ref, b_ref, o_ref):
    o_ref[...] = lax.dot_general(
        w_ref[...], z_ref[...],
        dimension_numbers=(((1,), (1,)), ((), ())),
        preferred_element_type=jnp.float32,
    ) + b_ref[...]


def _tc_project_t(z, out_w, out_b, tile):
    B, D = z.shape
    V, _ = out_w.shape
    out_t = pl.pallas_call(
        _mm_t_body,
        grid=(V // tile,),
        in_specs=[
            pl.BlockSpec((B, D), lambda i: (0, 0)),
            pl.BlockSpec((tile, D), lambda i: (i, 0)),
            pl.BlockSpec((tile, 1), lambda i: (i, 0)),
        ],
        out_specs=pl.BlockSpec((tile, B), lambda i: (i, 0)),
        out_shape=jax.ShapeDtypeStruct((V, B), jnp.float32),
    )(z, out_w, out_b.reshape(V, 1))
    return out_t.T


def kernel(x, emb_table, out_w, out_b):
    z = _sc_gather(x.astype(jnp.int32), emb_table)
    return _tc_project_t(z, out_w, out_b, tile=2000)


# speedup vs baseline: 2.3914x; 1.0186x over previous
import functools

import jax
import jax.numpy as jnp
from jax import lax
from jax.experimental import pallas as pl
from jax.experimental.pallas import tpu as pltpu
from jax.experimental.pallas import tpu_sc as plsc


def _sc_gather(idx, table):
    B, = idx.shape
    V, D = table.shape
    info = plsc.get_sparse_core_info()
    NC, NS = info.num_cores, info.num_subcores
    NW = NC * NS
    b_per_w = B // NW
    mesh = plsc.VectorSubcoreMesh(core_axis_name="c", subcore_axis_name="s")

    @functools.partial(
        pl.kernel,
        mesh=mesh,
        out_type=jax.ShapeDtypeStruct((B, D), table.dtype),
        scratch_types=[
            pltpu.VMEM((b_per_w,), jnp.int32),
            pltpu.VMEM((b_per_w, D), table.dtype),
            pltpu.SemaphoreType.DMA,
        ],
    )
    def k(idx_hbm, table_hbm, out_hbm, idx_v, rows_v, sem):
        wid = lax.axis_index("s") * NC + lax.axis_index("c")
        base = wid * b_per_w
        pltpu.sync_copy(idx_hbm.at[pl.ds(base, b_per_w)], idx_v)
        pltpu.async_copy(table_hbm.at[idx_v], rows_v, sem).wait()
        pltpu.sync_copy(rows_v, out_hbm.at[pl.ds(base, b_per_w)])

    return k(idx, table)


def _mm_t_body(z_ref, w_ref, b_ref, o_ref):
    o_ref[...] = lax.dot_general(
        w_ref[...], z_ref[...],
        dimension_numbers=(((1,), (1,)), ((), ())),
        preferred_element_type=jnp.float32,
    ) + b_ref[...]


def _tc_project_t(z, out_w, out_b, tile):
    B, D = z.shape
    V, _ = out_w.shape
    out_t = pl.pallas_call(
        _mm_t_body,
        grid=(V // tile,),
        in_specs=[
            pl.BlockSpec((B, D), lambda i: (0, 0)),
            pl.BlockSpec((tile, D), lambda i: (i, 0)),
            pl.BlockSpec((tile, 1), lambda i: (i, 0)),
        ],
        out_specs=pl.BlockSpec((tile, B), lambda i: (i, 0)),
        out_shape=jax.ShapeDtypeStruct((V, B), jnp.float32),
    )(z, out_w, out_b.reshape(V, 1))
    return out_t.T


def kernel(x, emb_table, out_w, out_b):
    z = _sc_gather(x.astype(jnp.int32), emb_table)
    return _tc_project_t(z, out_w, out_b, tile=5000)


# speedup vs baseline: 3.2089x; 1.3418x over previous
import functools

import jax
import jax.numpy as jnp
from jax import lax
from jax.experimental import pallas as pl
from jax.experimental.pallas import tpu as pltpu
from jax.experimental.pallas import tpu_sc as plsc


def _sc_gather(idx, table):
    B, = idx.shape
    V, D = table.shape
    info = plsc.get_sparse_core_info()
    NC, NS = info.num_cores, info.num_subcores
    NW = NC * NS
    b_per_w = B // NW
    mesh = plsc.VectorSubcoreMesh(core_axis_name="c", subcore_axis_name="s")

    @functools.partial(
        pl.kernel,
        mesh=mesh,
        out_type=jax.ShapeDtypeStruct((B, D), table.dtype),
        scratch_types=[
            pltpu.VMEM((b_per_w,), jnp.int32),
            pltpu.VMEM((b_per_w, D), table.dtype),
            pltpu.SemaphoreType.DMA,
        ],
    )
    def k(idx_hbm, table_hbm, out_hbm, idx_v, rows_v, sem):
        wid = lax.axis_index("s") * NC + lax.axis_index("c")
        base = wid * b_per_w
        pltpu.sync_copy(idx_hbm.at[pl.ds(base, b_per_w)], idx_v)
        pltpu.async_copy(table_hbm.at[idx_v], rows_v, sem).wait()
        pltpu.sync_copy(rows_v, out_hbm.at[pl.ds(base, b_per_w)])

    return k(idx, table)


def _tc_project_t(z, out_w, out_b, tile):
    B, D = z.shape
    V, _ = out_w.shape

    def body(z_ref, w_ref, b_ref, o_ref):
        b_col = jnp.transpose(b_ref[...], (1, 0))
        o_ref[...] = lax.dot_general(
            w_ref[...], z_ref[...],
            dimension_numbers=(((1,), (1,)), ((), ())),
            preferred_element_type=jnp.float32,
        ) + b_col

    out_t = pl.pallas_call(
        body,
        grid=(pl.cdiv(V, tile),),
        in_specs=[
            pl.BlockSpec((B, D), lambda i: (0, 0)),
            pl.BlockSpec((tile, D), lambda i: (i, 0)),
            pl.BlockSpec((1, tile), lambda i: (0, i)),
        ],
        out_specs=pl.BlockSpec((tile, B), lambda i: (i, 0)),
        out_shape=jax.ShapeDtypeStruct((V, B), jnp.float32),
    )(z, out_w, out_b.reshape(1, V))
    return out_t.T


def kernel(x, emb_table, out_w, out_b):
    z = _sc_gather(x.astype(jnp.int32), emb_table)
    return _tc_project_t(z, out_w, out_b, tile=4096)
